# Initial kernel scaffold; baseline (speedup 1.0000x reference)
import jax
import jax.numpy as jnp
from jax.experimental import pallas as pl


def kernel(x, edge_index, L_vals, weight, bias):
    raise NotImplementedError("write your pallas kernel here")


# speedup vs baseline: 4.0284x; 4.0284x over previous
import dataclasses
import functools

import jax
import jax.numpy as jnp
from jax import lax
from jax.experimental import pallas as pl
from jax.experimental.pallas import tpu as pltpu
from jax.experimental.pallas import tpu_sc as plsc

NC = 2
NS = 16
LN = 16


def _sc_spmm(src, dst, lvals, x):
    E = src.shape[0]
    N, D = x.shape
    NW = NC * NS
    EPW = E // NW
    C = 80
    assert EPW * NW == E and EPW % C == 0 and D % LN == 0
    NP = ((N + 128 * NS - 1) // (128 * NS)) * (128 * NS)
    RPT = NP // NS
    ZR = 128
    assert RPT % ZR == 0

    mesh = plsc.VectorSubcoreMesh(
        core_axis_name="c", subcore_axis_name="s", num_cores=NC, num_subcores=NS
    )
    cp = pltpu.CompilerParams()
    if "needs_layout_passes" in pltpu.CompilerParams.__dataclass_fields__:
        cp = dataclasses.replace(cp, needs_layout_passes=False)

    @functools.partial(
        pl.kernel,
        out_type=jax.ShapeDtypeStruct((NC, NP, D), jnp.float32),
        mesh=mesh,
        compiler_params=cp,
        scratch_types=[
            pltpu.VMEM_SHARED((NP, D), jnp.float32),
            pltpu.VMEM((C,), jnp.int32),
            pltpu.VMEM((C,), jnp.int32),
            pltpu.VMEM((C,), jnp.float32),
            pltpu.VMEM((C, D), jnp.float32),
            pltpu.VMEM((ZR, D), jnp.float32),
        ],
    )
    def sc_kernel(src_hbm, dst_hbm, lv_hbm, x_hbm, part_hbm,
                  acc, src_v, dst_v, lv_v, rows_v, zbuf):
        cid = lax.axis_index("c")
        sid = lax.axis_index("s")
        wid = sid * NC + cid

        zero16 = jnp.zeros((LN,), jnp.float32)

        @pl.loop(0, ZR)
        def _(r):
            for k in range(D // LN):
                zbuf[r, pl.ds(k * LN, LN)] = zero16

        row0 = sid * RPT
        for t in range(RPT // ZR):
            pltpu.sync_copy(zbuf, acc.at[pl.ds(row0 + t * ZR, ZR)])
        plsc.subcore_barrier()

        base_e = wid * EPW

        @pl.loop(0, EPW, step=C)
        def _(e0):
            b = base_e + e0
            pltpu.sync_copy(src_hbm.at[pl.ds(b, C)], src_v)
            pltpu.sync_copy(dst_hbm.at[pl.ds(b, C)], dst_v)
            pltpu.sync_copy(lv_hbm.at[pl.ds(b, C)], lv_v)
            pltpu.sync_copy(x_hbm.at[src_v], rows_v)

            @pl.loop(0, C)
            def _(e):
                lval = plsc.load_gather(lv_v, [jnp.full((LN,), e, jnp.int32)])
                for k in range(D // LN):
                    sl = pl.ds(k * LN, LN)
                    rows_v[e, sl] = rows_v[e, sl] * lval

            pltpu.sync_copy(rows_v, acc.at[dst_v], add=True)

        plsc.subcore_barrier()

        for t in range(RPT // ZR):
            r = row0 + t * ZR
            pltpu.sync_copy(acc.at[pl.ds(r, ZR)], part_hbm.at[cid, pl.ds(r, ZR)])

    return sc_kernel(src, dst, lvals, x)


def _tc_finish(partials, weight, bias, N):
    D_IN, D_OUT = weight.shape
    BN = 400
    assert N % BN == 0

    def body(p_ref, w_ref, b_ref, o_ref):
        s = p_ref[0] + p_ref[1]
        y = jnp.dot(s, w_ref[...], preferred_element_type=jnp.float32)
        y = y + b_ref[...]
        o_ref[...] = jnp.where(y >= 0, y, 0.01 * y)

    return pl.pallas_call(
        body,
        grid=(N // BN,),
        in_specs=[
            pl.BlockSpec((2, BN, D_IN), lambda i: (0, i, 0)),
            pl.BlockSpec((D_IN, D_OUT), lambda i: (0, 0)),
            pl.BlockSpec((1, D_OUT), lambda i: (0, 0)),
        ],
        out_specs=pl.BlockSpec((BN, D_OUT), lambda i: (i, 0)),
        out_shape=jax.ShapeDtypeStruct((N, D_OUT), jnp.float32),
    )(partials, weight, bias)


def kernel(x, edge_index, L_vals, weight, bias):
    dst = edge_index[0]
    src = edge_index[1]
    partials = _sc_spmm(src, dst, L_vals, x)
    return _tc_finish(partials, weight, bias.reshape(1, -1), x.shape[0])


# speedup vs baseline: 8.6349x; 2.1435x over previous
import dataclasses
import functools

import jax
import jax.numpy as jnp
from jax import lax
from jax.experimental import pallas as pl
from jax.experimental.pallas import tpu as pltpu
from jax.experimental.pallas import tpu_sc as plsc

NC = 2
NS = 16
LN = 16


def _sc_spmm(src, dst, lvals, x):
    E = src.shape[0]
    N, D = x.shape
    NW = NC * NS
    EPW = E // NW
    C = 80
    assert EPW * NW == E and EPW % C == 0 and D % LN == 0
    assert (EPW // C) % 2 == 1
    KB = 25
    EB = KB * C
    NB = EPW // EB
    assert EPW % EB == 0 and KB % 2 == 1
    NP = ((N + 128 * NS - 1) // (128 * NS)) * (128 * NS)
    RPT = NP // NS
    ZR = C
    assert RPT % ZR == 0

    mesh = plsc.VectorSubcoreMesh(
        core_axis_name="c", subcore_axis_name="s", num_cores=NC, num_subcores=NS
    )
    cp = pltpu.CompilerParams()
    if "needs_layout_passes" in pltpu.CompilerParams.__dataclass_fields__:
        cp = dataclasses.replace(cp, needs_layout_passes=False)

    @functools.partial(
        pl.kernel,
        out_type=jax.ShapeDtypeStruct((NC, NP, D), jnp.float32),
        mesh=mesh,
        compiler_params=cp,
        scratch_types=[
            pltpu.VMEM_SHARED((NP, D), jnp.float32),
            pltpu.VMEM((EB,), jnp.int32),
            pltpu.VMEM((EB,), jnp.int32),
            pltpu.VMEM((EB,), jnp.float32),
            pltpu.VMEM((C,), jnp.int32),
            pltpu.VMEM((C, D), jnp.float32),
            pltpu.VMEM((C, D), jnp.float32),
            pltpu.SemaphoreType.DMA,
            pltpu.SemaphoreType.DMA,
        ],
    )
    def sc_kernel(src_hbm, dst_hbm, lv_hbm, x_hbm, part_hbm,
                  acc, src_i, dst_i, lv_i, dst_v, buf_a, buf_b,
                  sem_a, sem_b):
        cid = lax.axis_index("c")
        sid = lax.axis_index("s")
        wid = sid * NC + cid

        zero16 = jnp.zeros((LN,), jnp.float32)

        @pl.loop(0, ZR)
        def _(r):
            for k in range(D // LN):
                buf_a[r, pl.ds(k * LN, LN)] = zero16

        row0 = sid * RPT
        for t in range(RPT // ZR):
            pltpu.sync_copy(buf_a, acc.at[pl.ds(row0 + t * ZR, ZR)])
        plsc.subcore_barrier()

        base_e = wid * EPW

        def gather(j, buf, sem):
            off = pl.multiple_of(j * C, C)
            return pltpu.make_async_copy(
                x_hbm.at[src_i.at[pl.ds(off, C)]], buf, sem)

        def process(j, buf):
            off = pl.multiple_of(j * C, C)
            for q in range(C // LN):
                dst_v[pl.ds(q * LN, LN)] = dst_i[pl.ds(off + q * LN, LN)]

            @pl.loop(0, C)
            def _(e):
                lval = plsc.load_gather(
                    lv_i, [jnp.full((LN,), off + e, jnp.int32)])
                for k in range(D // LN):
                    sl = pl.ds(k * LN, LN)
                    buf[e, sl] = buf[e, sl] * lval

            pltpu.sync_copy(buf, acc.at[dst_v], add=True)

        @pl.loop(0, NB)
        def _(b):
            eb = base_e + b * EB
            pltpu.sync_copy(src_hbm.at[pl.ds(eb, EB)], src_i)
            pltpu.sync_copy(dst_hbm.at[pl.ds(eb, EB)], dst_i)
            pltpu.sync_copy(lv_hbm.at[pl.ds(eb, EB)], lv_i)

            gather(0, buf_a, sem_a).start()

            @pl.loop(0, KB - 1, step=2)
            def _(j):
                gather(j + 1, buf_b, sem_b).start()
                gather(j, buf_a, sem_a).wait()
                process(j, buf_a)
                gather(j + 2, buf_a, sem_a).start()
                gather(j + 1, buf_b, sem_b).wait()
                process(j + 1, buf_b)

            gather(KB - 1, buf_a, sem_a).wait()
            process(KB - 1, buf_a)

        plsc.subcore_barrier()

        for t in range(RPT // ZR):
            r = row0 + t * ZR
            pltpu.sync_copy(acc.at[pl.ds(r, ZR)], part_hbm.at[cid, pl.ds(r, ZR)])

    return sc_kernel(src, dst, lvals, x)


def _tc_finish(partials, weight, bias, N):
    D_IN, D_OUT = weight.shape
    BN = 400
    assert N % BN == 0

    def body(p_ref, w_ref, b_ref, o_ref):
        s = p_ref[0] + p_ref[1]
        y = jnp.dot(s, w_ref[...], preferred_element_type=jnp.float32)
        y = y + b_ref[...]
        o_ref[...] = jnp.where(y >= 0, y, 0.01 * y)

    return pl.pallas_call(
        body,
        grid=(N // BN,),
        in_specs=[
            pl.BlockSpec((2, BN, D_IN), lambda i: (0, i, 0)),
            pl.BlockSpec((D_IN, D_OUT), lambda i: (0, 0)),
            pl.BlockSpec((1, D_OUT), lambda i: (0, 0)),
        ],
        out_specs=pl.BlockSpec((BN, D_OUT), lambda i: (i, 0)),
        out_shape=jax.ShapeDtypeStruct((N, D_OUT), jnp.float32),
    )(partials, weight, bias)


def kernel(x, edge_index, L_vals, weight, bias):
    dst = edge_index[0]
    src = edge_index[1]
    partials = _sc_spmm(src, dst, L_vals, x)
    return _tc_finish(partials, weight, bias.reshape(1, -1), x.shape[0])
